# baseline (speedup 1.0000x reference)
import functools

import jax
import jax.numpy as jnp
from jax import lax
from jax.experimental import pallas as pl
from jax.experimental.pallas import tpu as pltpu

_N_BINS = 15
_LOG2E = 1.4426950408889634
_BLK = 4000


def _ece_kernel(logits_hbm, lab_hbm, out_ref, xbuf, lbuf, acc_ref,
                sem_x, sem_l, *, n_total, n_blocks, blk):
    step = pl.program_id(0)
    slot = lax.rem(step, 2)
    nslot = lax.rem(step + 1, 2)

    @pl.when(step == 0)
    def _init():
        acc_ref[...] = jnp.zeros_like(acc_ref)
        pltpu.make_async_copy(
            logits_hbm.at[pl.ds(0, blk), :], xbuf.at[0], sem_x.at[0]).start()
        pltpu.make_async_copy(
            lab_hbm.at[0], lbuf.at[0], sem_l.at[0]).start()

    @pl.when(step + 1 < n_blocks)
    def _prefetch():
        pltpu.make_async_copy(
            logits_hbm.at[pl.ds((step + 1) * blk, blk), :],
            xbuf.at[nslot], sem_x.at[nslot]).start()
        pltpu.make_async_copy(
            lab_hbm.at[step + 1], lbuf.at[nslot], sem_l.at[nslot]).start()

    pltpu.make_async_copy(
        logits_hbm.at[pl.ds(step * blk, blk), :],
        xbuf.at[slot], sem_x.at[slot]).wait()
    pltpu.make_async_copy(
        lab_hbm.at[step], lbuf.at[slot], sem_l.at[slot]).wait()

    x = xbuf[slot]
    lab_row = lbuf[slot]
    b, c = x.shape

    lab = jnp.transpose(lab_row)
    idx = lax.broadcasted_iota(jnp.int32, (b, c), 1)
    onehot = (idx == lab)
    m = jnp.max(x, axis=1, keepdims=True)
    s = jnp.sum(jnp.exp2((x - m) * _LOG2E), axis=1, keepdims=True)
    g = jnp.sum(jnp.where(onehot, x, 0.0), axis=1, keepdims=True)
    conf = 1.0 / s
    acc = (g == m).astype(jnp.float32)

    ii = lax.broadcasted_iota(jnp.int32, (1, _N_BINS), 1).astype(jnp.float32)
    lo = ii / _N_BINS
    hi = (ii + 1.0) / _N_BINS
    mask = ((conf > lo) & (conf <= hi)).astype(jnp.float32)
    acc_ref[0, :] += jnp.sum(mask, axis=0)
    acc_ref[1, :] += jnp.sum(conf * mask, axis=0)
    acc_ref[2, :] += jnp.sum(acc * mask, axis=0)

    @pl.when(step == n_blocks - 1)
    def _finish():
        cnt = acc_ref[0, :]
        safe = jnp.maximum(cnt, 1.0)
        avg_conf = acc_ref[1, :] / safe
        avg_acc = acc_ref[2, :] / safe
        prop = cnt / n_total
        contrib = jnp.abs(avg_conf - avg_acc) * prop
        out_ref[...] = jnp.sum(jnp.where(prop > 0, contrib, 0.0)).reshape(1, 1)


def kernel(logits, labels):
    n, c = logits.shape
    blk = _BLK
    n_blocks = n // blk
    labels2 = labels.astype(jnp.int32).reshape(n_blocks, 1, blk)
    out = pl.pallas_call(
        functools.partial(_ece_kernel, n_total=float(n), n_blocks=n_blocks,
                          blk=blk),
        grid=(n_blocks,),
        in_specs=[
            pl.BlockSpec(memory_space=pltpu.MemorySpace.HBM),
            pl.BlockSpec(memory_space=pltpu.MemorySpace.HBM),
        ],
        out_specs=pl.BlockSpec((1, 1), lambda i: (0, 0)),
        out_shape=jax.ShapeDtypeStruct((1, 1), jnp.float32),
        scratch_shapes=[
            pltpu.VMEM((2, blk, c), jnp.float32),
            pltpu.VMEM((2, 1, blk), jnp.int32),
            pltpu.VMEM((3, _N_BINS), jnp.float32),
            pltpu.SemaphoreType.DMA((2,)),
            pltpu.SemaphoreType.DMA((2,)),
        ],
        compiler_params=pltpu.CompilerParams(
            dimension_semantics=("arbitrary",)),
    )(logits, labels2)
    return out.reshape(1)
